# baseline (speedup 1.0000x reference)
import jax
import jax.numpy as jnp
from jax import lax
from jax.experimental import pallas as pl
from jax.experimental.pallas import tpu as pltpu
from jax.experimental.pallas import tpu_sc as plsc

B = 16384
L = 20
EMB = 32
TEXT_V = 10000
NC = 2
NS = 16
NW = NC * NS
BPW = B // NW
C = 32
NCHUNK = BPW // C
NGROUP = BPW // 16
VPT = TEXT_V // NS


def _body(title_hbm, tokT_hbm, item_hbm, text_hbm, out_hbm,
          ids_v, idp_v, idm_v, id_rows, tok_v, rows_v, inv_v, n0_v, out_c,
          t0_v, text_sh, sem, sem2):
    cid = lax.axis_index("c")
    sid = lax.axis_index("s")
    wid = sid * NC + cid
    base = wid * BPW
    iota = lax.iota(jnp.int32, 16)

    pltpu.sync_copy(title_hbm.at[pl.ds(base, BPW)], ids_v)
    pltpu.sync_copy(tokT_hbm.at[:, pl.ds(base, BPW)], tok_v)

    for u in range(BPW // 16):
        idv = ids_v[pl.ds(u * 16, 16)]
        idp_v[pl.ds(u * 16, 16)] = lax.shift_right_logical(idv, 2)
        idm_v[pl.ds(u * 16, 16)] = (idv & 3) * 32

    b1 = pltpu.async_copy(item_hbm.at[idp_v], id_rows, sem2)

    vbase = sid * VPT
    for k in range(VPT // C):
        pltpu.sync_copy(text_hbm.at[pl.ds(vbase + k * C, C)], out_c)
        pltpu.sync_copy(out_c, text_sh.at[pl.ds(vbase + k * C, C)])
    _rem = VPT % C
    if _rem:
        pltpu.sync_copy(text_hbm.at[pl.ds(vbase + VPT - _rem, _rem)],
                        out_c.at[pl.ds(0, _rem)])
        pltpu.sync_copy(out_c.at[pl.ds(0, _rem)],
                        text_sh.at[pl.ds(vbase + VPT - _rem, _rem)])

    pltpu.sync_copy(text_hbm.at[pl.ds(0, 1)], t0_v)

    @pl.loop(0, NGROUP)
    def _count(g):
        off = g * 16
        n0i = jnp.zeros((16,), jnp.int32)
        for l in range(L):
            t = tok_v[l, pl.ds(off, 16)]
            n0i = n0i + jnp.where(t == 0, 1, 0)
        n0f = n0i.astype(jnp.float32)
        cnt = jnp.float32(L) - n0f
        inv = jnp.float32(1.0) / jnp.maximum(cnt, jnp.float32(1.0))
        n0_v[pl.ds(off, 16)] = n0f
        inv_v[pl.ds(off, 16)] = inv

    plsc.subcore_barrier()

    def _fire(cbase, p):
        for l in range(L):
            pltpu.async_copy(
                text_sh.at[tok_v.at[l, pl.ds(cbase, C)]],
                rows_v.at[p, l], sem)

    def _wait(cbase, p):
        for l in range(L):
            pltpu.make_async_copy(
                text_sh.at[tok_v.at[l, pl.ds(cbase, C)]],
                rows_v.at[p, l], sem).wait()

    def _compute(cbase, p):
        for r in range(C):
            bl = cbase + r
            lane_b = jnp.full((16,), bl, jnp.int32)
            n0b = plsc.load_gather(n0_v, [lane_b])
            invb = plsc.load_gather(inv_v, [lane_b])
            for h in range(2):
                vals = [rows_v[p, l, r, pl.ds(h * 16, 16)] for l in range(L)]
                while len(vals) > 1:
                    nxt = [vals[i] + vals[i + 1]
                           for i in range(0, len(vals) - 1, 2)]
                    if len(vals) % 2:
                        nxt.append(vals[-1])
                    vals = nxt
                s = vals[0]
                t0h = t0_v[0, pl.ds(h * 16, 16)]
                out_c[r, pl.ds(h * 16, 16)] = (s - n0b * t0h) * invb
        pltpu.sync_copy(out_c,
                        out_hbm.at[pl.ds(base + cbase, C), pl.ds(EMB, EMB)])

    @pl.loop(0, NCHUNK)
    def _chunk(c):
        cbase = c * C
        _fire(cbase, 0)
        _wait(cbase, 0)
        _compute(cbase, 0)

    b1.wait()

    @pl.loop(0, NCHUNK)
    def _bx(g):
        gbase = g * C
        for r in range(C):
            rr = gbase + r
            idmb = plsc.load_gather(idm_v, [jnp.full((16,), rr, jnp.int32)])
            for h in range(2):
                col16 = idmb + h * 16 + iota
                out_c[r, pl.ds(h * 16, 16)] = plsc.load_gather(
                    id_rows, [jnp.full((16,), rr, jnp.int32), col16])
        pltpu.sync_copy(out_c,
                        out_hbm.at[pl.ds(base + gbase, C), pl.ds(0, EMB)])


PCH = 782
ITEM_V = 100001
PACK_ROWS = PCH * 32


def _repack_body(itemT_hbm, pk_hbm, blk_v, out_v, sem):
    cid = lax.axis_index("c")
    sid = lax.axis_index("s")
    wid = sid * NC + cid
    iota = lax.iota(jnp.int32, 16)
    per = PCH // NW + 1

    @pl.loop(0, per)
    def _rep(t):
        tc = wid * per + t

        @pl.when(tc < PCH)
        def _do():
            pltpu.sync_copy(itemT_hbm.at[:, pl.ds(tc * 128, 128)], blk_v)
            for pr in range(32):
                for h in range(8):
                    q = (h * 16) // 32
                    d16 = (h * 16) % 32 + iota
                    c = jnp.full((16,), 4 * pr + q, jnp.int32)
                    out_v[pr, pl.ds(h * 16, 16)] = plsc.load_gather(
                        blk_v, [d16, c])
            pltpu.sync_copy(out_v, pk_hbm.at[pl.ds(tc * 32, 32), :])


_repack = pl.kernel(
    _repack_body,
    out_type=jax.ShapeDtypeStruct((PACK_ROWS, 128), jnp.float32),
    mesh=plsc.VectorSubcoreMesh(core_axis_name="c", subcore_axis_name="s"),
    compiler_params=pltpu.CompilerParams(needs_layout_passes=False),
    scratch_types=[
        pltpu.VMEM((32, 128), jnp.float32),
        pltpu.VMEM((32, 128), jnp.float32),
        pltpu.SemaphoreType.DMA,
    ],
)

_mesh = plsc.VectorSubcoreMesh(core_axis_name="c", subcore_axis_name="s")

_sc_call = pl.kernel(
    _body,
    out_type=jax.ShapeDtypeStruct((B, 2 * EMB), jnp.float32),
    mesh=_mesh,
    compiler_params=pltpu.CompilerParams(use_tc_tiling_on_sc=False,
                                         needs_layout_passes=False),
    scratch_types=[
        pltpu.VMEM((BPW,), jnp.int32),
        pltpu.VMEM((BPW,), jnp.int32),
        pltpu.VMEM((BPW,), jnp.int32),
        pltpu.VMEM((BPW, 128), jnp.float32),
        pltpu.VMEM((L, BPW), jnp.int32),
        pltpu.VMEM((1, L, C, EMB), jnp.float32),
        pltpu.VMEM((BPW,), jnp.float32),
        pltpu.VMEM((BPW,), jnp.float32),
        pltpu.VMEM((C, EMB), jnp.float32),
        pltpu.VMEM((1, EMB), jnp.float32),
        pltpu.VMEM_SHARED((TEXT_V, EMB), jnp.float32),
        pltpu.SemaphoreType.DMA,
        pltpu.SemaphoreType.DMA,
    ],
)


def kernel(title_ids, title_token_ids, item_table, text_table):
    tokT = title_token_ids.T
    item_pk = _repack(item_table.T)
    return _sc_call(title_ids, tokT, item_pk, text_table)


# speedup vs baseline: 1.5764x; 1.5764x over previous
import jax
import jax.numpy as jnp
from jax import lax
from jax.experimental import pallas as pl
from jax.experimental.pallas import tpu as pltpu
from jax.experimental.pallas import tpu_sc as plsc

B = 16384
L = 20
EMB = 32
TEXT_V = 10000
NC = 2
NS = 16
NW = NC * NS
BPW = B // NW
C = 32
NCHUNK = BPW // C
NGROUP = BPW // 16
VPT = TEXT_V // NS


def _body(title_hbm, tokT_hbm, item_hbm, text_hbm, out_hbm,
          ids_v, id_rows, tok_v, rows_v, stage_v, inv_v, n0_v, out_c, t0_v,
          text_sh, sem, sem2):
    cid = lax.axis_index("c")
    sid = lax.axis_index("s")
    wid = sid * NC + cid
    base = wid * BPW

    pltpu.sync_copy(title_hbm.at[pl.ds(base, BPW)], ids_v)
    pltpu.sync_copy(tokT_hbm.at[:, pl.ds(base, BPW)], tok_v)

    b1 = pltpu.async_copy(item_hbm.at[ids_v], id_rows, sem2)

    vbase = sid * VPT
    pltpu.sync_copy(text_hbm.at[pl.ds(vbase, VPT)], stage_v)
    pltpu.sync_copy(stage_v, text_sh.at[pl.ds(vbase, VPT)])

    pltpu.sync_copy(text_hbm.at[pl.ds(0, 1)], t0_v)

    @pl.loop(0, NGROUP)
    def _count(g):
        off = g * 16
        n0i = jnp.zeros((16,), jnp.int32)
        for l in range(L):
            t = tok_v[l, pl.ds(off, 16)]
            n0i = n0i + jnp.where(t == 0, 1, 0)
        n0f = n0i.astype(jnp.float32)
        cnt = jnp.float32(L) - n0f
        inv = jnp.float32(1.0) / jnp.maximum(cnt, jnp.float32(1.0))
        n0_v[pl.ds(off, 16)] = n0f
        inv_v[pl.ds(off, 16)] = inv

    plsc.subcore_barrier()

    def _fire(cbase, p):
        for l in range(L):
            pltpu.async_copy(
                text_sh.at[tok_v.at[l, pl.ds(cbase, C)]],
                rows_v.at[p, l], sem)

    def _wait(cbase, p):
        for l in range(L):
            pltpu.make_async_copy(
                text_sh.at[tok_v.at[l, pl.ds(cbase, C)]],
                rows_v.at[p, l], sem).wait()

    def _compute(cbase, p):
        for r in range(C):
            bl = cbase + r
            lane_b = jnp.full((16,), bl, jnp.int32)
            n0b = plsc.load_gather(n0_v, [lane_b])
            invb = plsc.load_gather(inv_v, [lane_b])
            for h in range(2):
                vals = [rows_v[p, l, r, pl.ds(h * 16, 16)] for l in range(L)]
                while len(vals) > 1:
                    nxt = [vals[i] + vals[i + 1]
                           for i in range(0, len(vals) - 1, 2)]
                    if len(vals) % 2:
                        nxt.append(vals[-1])
                    vals = nxt
                s = vals[0]
                t0h = t0_v[0, pl.ds(h * 16, 16)]
                out_c[r, pl.ds(h * 16, 16)] = (s - n0b * t0h) * invb
        pltpu.sync_copy(out_c,
                        out_hbm.at[pl.ds(base + cbase, C), pl.ds(EMB, EMB)])

    _fire(0, 0)

    @pl.loop(0, NCHUNK, step=2)
    def _chunk(c):
        cbase = c * C
        _fire(cbase + C, 1)
        _wait(cbase, 0)
        _compute(cbase, 0)
        _fire(lax.rem(cbase + 2 * C, NCHUNK * C), 0)
        _wait(cbase + C, 1)
        _compute(cbase + C, 1)

    _wait(0, 0)

    b1.wait()
    pltpu.sync_copy(id_rows, out_hbm.at[pl.ds(base, BPW), pl.ds(0, EMB)])


_mesh = plsc.VectorSubcoreMesh(core_axis_name="c", subcore_axis_name="s")

_sc_call = pl.kernel(
    _body,
    out_type=jax.ShapeDtypeStruct((B, 2 * EMB), jnp.float32),
    mesh=_mesh,
    compiler_params=pltpu.CompilerParams(use_tc_tiling_on_sc=False,
                                         needs_layout_passes=False),
    scratch_types=[
        pltpu.VMEM((BPW,), jnp.int32),
        pltpu.VMEM((BPW, EMB), jnp.float32),
        pltpu.VMEM((L, BPW), jnp.int32),
        pltpu.VMEM((2, L, C, EMB), jnp.float32),
        pltpu.VMEM((VPT, EMB), jnp.float32),
        pltpu.VMEM((BPW,), jnp.float32),
        pltpu.VMEM((BPW,), jnp.float32),
        pltpu.VMEM((C, EMB), jnp.float32),
        pltpu.VMEM((1, EMB), jnp.float32),
        pltpu.VMEM_SHARED((TEXT_V, EMB), jnp.float32),
        pltpu.SemaphoreType.DMA,
        pltpu.SemaphoreType.DMA,
    ],
)


def kernel(title_ids, title_token_ids, item_table, text_table):
    tokT = title_token_ids.T
    return _sc_call(title_ids, tokT, item_table, text_table)
